# baseline (speedup 1.0000x reference)
import jax
import jax.numpy as jnp
from jax.experimental import pallas as pl

PASS = 8
FLOW = 16
TRANS = FLOW - PASS
NBINS = 64
HID = 64
BLOCK = 2048
BIG = 3.0e38


def _coupling_kernel(y_ref, W1T_ref, b1_ref, W2T_ref, b2_ref, W3T_ref,
                     tric_ref, out_ref):
    yT = y_ref[...].T
    yAT = yT[:PASS, :]

    h = jnp.maximum(
        jnp.dot(W1T_ref[...], yAT, preferred_element_type=jnp.float32)
        + b1_ref[...], 0.0)
    h = jnp.maximum(
        jnp.dot(W2T_ref[...], h, preferred_element_type=jnp.float32)
        + b2_ref[...], 0.0)
    logits = jnp.dot(W3T_ref[...], h, preferred_element_type=jnp.float32)
    E = jnp.exp(logits)

    rows = [yAT]
    inv_prod = yT[FLOW:FLOW + 1, :]
    for t in range(TRANS):
        Et = E[t * NBINS:(t + 1) * NBINS, :]
        Qs = jnp.dot(tric_ref[t * NBINS:(t + 1) * NBINS, :], Et,
                     preferred_element_type=jnp.float32)
        Qnorm = Qs[NBINS - 1:NBINS, :]
        u = yT[PASS + t:PASS + t + 1, :] * Qnorm
        Qs63 = Qs[:NBINS - 1, :]
        lt = Qs63 < u
        ybins = jnp.sum(lt.astype(jnp.float32), axis=0, keepdims=True)
        offset = jnp.max(jnp.where(lt, Qs63, 0.0), axis=0, keepdims=True)
        nxt = jnp.min(jnp.where(lt, BIG, Qs63), axis=0, keepdims=True)
        nxt = jnp.minimum(nxt, Qnorm)
        slope64 = (nxt - offset) * float(NBINS)
        rows.append((u - offset) / slope64 + ybins * (1.0 / NBINS))
        inv_prod = inv_prod * (Qnorm / slope64)

    rows.append(inv_prod)
    out_ref[...] = jnp.concatenate(rows, axis=0).T


def kernel(y, W1, b1, W2, b2, W3, b3):
    B = y.shape[0]
    grid = (B // BLOCK,)
    tri = (jnp.arange(NBINS)[:, None] >= jnp.arange(NBINS)[None, :]
           ).astype(jnp.float32)
    tric = (tri[None, :, :] * jnp.exp(b3).reshape(TRANS, 1, NBINS)
            ).reshape(TRANS * NBINS, NBINS)
    return pl.pallas_call(
        _coupling_kernel,
        grid=grid,
        in_specs=[
            pl.BlockSpec((BLOCK, FLOW + 1), lambda i: (i, 0)),
            pl.BlockSpec((HID, PASS), lambda i: (0, 0)),
            pl.BlockSpec((HID, 1), lambda i: (0, 0)),
            pl.BlockSpec((HID, HID), lambda i: (0, 0)),
            pl.BlockSpec((HID, 1), lambda i: (0, 0)),
            pl.BlockSpec((TRANS * NBINS, HID), lambda i: (0, 0)),
            pl.BlockSpec((TRANS * NBINS, NBINS), lambda i: (0, 0)),
        ],
        out_specs=pl.BlockSpec((BLOCK, FLOW + 1), lambda i: (i, 0)),
        out_shape=jax.ShapeDtypeStruct((B, FLOW + 1), jnp.float32),
    )(y, W1.T, b1[:, None], W2.T, b2[:, None], W3.T, tric)


# speedup vs baseline: 1.4810x; 1.4810x over previous
import jax
import jax.numpy as jnp
from jax.experimental import pallas as pl

PASS = 8
FLOW = 16
TRANS = FLOW - PASS
NBINS = 64
HID = 64
BLOCK = 4096
BIG = 3.0e38


def _coupling_kernel(yT_ref, W1T_ref, b1_ref, W2T_ref, b2_ref, W3T_ref,
                     tric_ref, out_ref):
    yT = yT_ref[...]
    yAT = yT[:PASS, :]

    h = jnp.maximum(
        jnp.dot(W1T_ref[...], yAT, preferred_element_type=jnp.float32)
        + b1_ref[...], 0.0)
    h = jnp.maximum(
        jnp.dot(W2T_ref[...], h, preferred_element_type=jnp.float32)
        + b2_ref[...], 0.0)
    logits = jnp.dot(W3T_ref[...], h, preferred_element_type=jnp.float32)
    E = jnp.exp(logits)

    inv_prod = yT[FLOW:FLOW + 1, :]
    for t in range(TRANS):
        Et = E[t * NBINS:(t + 1) * NBINS, :]
        Qs = jnp.dot(tric_ref[t * NBINS:(t + 1) * NBINS, :], Et,
                     preferred_element_type=jnp.float32)
        Qnorm = Qs[NBINS - 1:NBINS, :]
        u = yT[PASS + t:PASS + t + 1, :] * Qnorm
        Qs63 = Qs[:NBINS - 1, :]
        lt = Qs63 < u
        ybins = jnp.sum(lt.astype(jnp.float32), axis=0, keepdims=True)
        offset = jnp.max(jnp.where(lt, Qs63, 0.0), axis=0, keepdims=True)
        nxt = jnp.min(jnp.where(lt, BIG, Qs63), axis=0, keepdims=True)
        nxt = jnp.minimum(nxt, Qnorm)
        slope64 = (nxt - offset) * float(NBINS)
        out_ref[t:t + 1, :] = (u - offset) / slope64 + ybins * (1.0 / NBINS)
        inv_prod = inv_prod * (Qnorm / slope64)

    out_ref[TRANS:TRANS + 1, :] = inv_prod


def kernel(y, W1, b1, W2, b2, W3, b3):
    B = y.shape[0]
    grid = (B // BLOCK,)
    tri = (jnp.arange(NBINS)[:, None] >= jnp.arange(NBINS)[None, :]
           ).astype(jnp.float32)
    tric = (tri[None, :, :] * jnp.exp(b3).reshape(TRANS, 1, NBINS)
            ).reshape(TRANS * NBINS, NBINS)
    o9 = pl.pallas_call(
        _coupling_kernel,
        grid=grid,
        in_specs=[
            pl.BlockSpec((FLOW + 1, BLOCK), lambda i: (0, i)),
            pl.BlockSpec((HID, PASS), lambda i: (0, 0)),
            pl.BlockSpec((HID, 1), lambda i: (0, 0)),
            pl.BlockSpec((HID, HID), lambda i: (0, 0)),
            pl.BlockSpec((HID, 1), lambda i: (0, 0)),
            pl.BlockSpec((TRANS * NBINS, HID), lambda i: (0, 0)),
            pl.BlockSpec((TRANS * NBINS, NBINS), lambda i: (0, 0)),
        ],
        out_specs=pl.BlockSpec((TRANS + 1, BLOCK), lambda i: (0, i)),
        out_shape=jax.ShapeDtypeStruct((TRANS + 1, B), jnp.float32),
    )(y.T, W1.T, b1[:, None], W2.T, b2[:, None], W3.T, tric)
    return jnp.concatenate([y[:, :PASS], o9.T], axis=1)


# speedup vs baseline: 1.4860x; 1.0033x over previous
import jax
import jax.numpy as jnp
from jax.experimental import pallas as pl

PASS = 8
FLOW = 16
TRANS = FLOW - PASS
NBINS = 64
HID = 64
BLOCK = 8192
BIG = 3.0e38


def _coupling_kernel(yT_ref, W1T_ref, b1_ref, W2T_ref, b2_ref, W3T_ref,
                     tric_ref, out_ref):
    yT = yT_ref[...]
    yAT = yT[:PASS, :]

    h = jnp.maximum(
        jnp.dot(W1T_ref[...], yAT, preferred_element_type=jnp.float32)
        + b1_ref[...], 0.0)
    h = jnp.maximum(
        jnp.dot(W2T_ref[...], h, preferred_element_type=jnp.float32)
        + b2_ref[...], 0.0)
    logits = jnp.dot(W3T_ref[...], h, preferred_element_type=jnp.float32)
    E = jnp.exp(logits)

    inv_prod = yT[FLOW:FLOW + 1, :]
    for t in range(TRANS):
        Et = E[t * NBINS:(t + 1) * NBINS, :]
        Qs = jnp.dot(tric_ref[t * NBINS:(t + 1) * NBINS, :], Et,
                     preferred_element_type=jnp.float32)
        Qnorm = Qs[NBINS - 1:NBINS, :]
        u = yT[PASS + t:PASS + t + 1, :] * Qnorm
        Qs63 = Qs[:NBINS - 1, :]
        lt = Qs63 < u
        ybins = jnp.sum(lt.astype(jnp.float32), axis=0, keepdims=True)
        offset = jnp.max(jnp.where(lt, Qs63, 0.0), axis=0, keepdims=True)
        nxt = jnp.min(jnp.where(lt, BIG, Qs63), axis=0, keepdims=True)
        nxt = jnp.minimum(nxt, Qnorm)
        slope64 = (nxt - offset) * float(NBINS)
        out_ref[t:t + 1, :] = (u - offset) / slope64 + ybins * (1.0 / NBINS)
        inv_prod = inv_prod * (Qnorm / slope64)

    out_ref[TRANS:TRANS + 1, :] = inv_prod


def kernel(y, W1, b1, W2, b2, W3, b3):
    B = y.shape[0]
    grid = (B // BLOCK,)
    tri = (jnp.arange(NBINS)[:, None] >= jnp.arange(NBINS)[None, :]
           ).astype(jnp.float32)
    tric = (tri[None, :, :] * jnp.exp(b3).reshape(TRANS, 1, NBINS)
            ).reshape(TRANS * NBINS, NBINS)
    o9 = pl.pallas_call(
        _coupling_kernel,
        grid=grid,
        in_specs=[
            pl.BlockSpec((FLOW + 1, BLOCK), lambda i: (0, i)),
            pl.BlockSpec((HID, PASS), lambda i: (0, 0)),
            pl.BlockSpec((HID, 1), lambda i: (0, 0)),
            pl.BlockSpec((HID, HID), lambda i: (0, 0)),
            pl.BlockSpec((HID, 1), lambda i: (0, 0)),
            pl.BlockSpec((TRANS * NBINS, HID), lambda i: (0, 0)),
            pl.BlockSpec((TRANS * NBINS, NBINS), lambda i: (0, 0)),
        ],
        out_specs=pl.BlockSpec((TRANS + 1, BLOCK), lambda i: (0, i)),
        out_shape=jax.ShapeDtypeStruct((TRANS + 1, B), jnp.float32),
    )(y.T, W1.T, b1[:, None], W2.T, b2[:, None], W3.T, tric)
    return jnp.concatenate([y[:, :PASS], o9.T], axis=1)


# speedup vs baseline: 1.5757x; 1.0604x over previous
import jax
import jax.numpy as jnp
from jax.experimental import pallas as pl

PASS = 8
FLOW = 16
TRANS = FLOW - PASS
NBINS = 64
HID = 64
BLOCK = 8192
BIG = 3.0e38


def _coupling_kernel(yT_ref, W1T_ref, b1_ref, W2T_ref, b2_ref, W3T_ref,
                     tric_ref, out_ref):
    yT = yT_ref[...]
    yAT = yT[:PASS, :]

    h = jnp.maximum(
        jnp.dot(W1T_ref[...], yAT, preferred_element_type=jnp.float32)
        + b1_ref[...], 0.0)
    h = jnp.maximum(
        jnp.dot(W2T_ref[...], h, preferred_element_type=jnp.float32)
        + b2_ref[...], 0.0)
    logits = jnp.dot(W3T_ref[...], h, preferred_element_type=jnp.float32)
    E = jnp.exp(logits)

    inv_prod = yT[FLOW:FLOW + 1, :]
    for t in range(TRANS):
        Et = E[t * NBINS:(t + 1) * NBINS, :]
        Qs = jnp.dot(tric_ref[t * NBINS:(t + 1) * NBINS, :], Et,
                     preferred_element_type=jnp.float32)
        Qnorm = Qs[NBINS - 1:NBINS, :]
        u = yT[PASS + t:PASS + t + 1, :] * Qnorm
        C32 = Qs[NBINS // 2 - 1:NBINS // 2, :]
        go = C32 < u
        half = jnp.where(go, Qs[NBINS // 2:NBINS - 1, :],
                         Qs[:NBINS // 2 - 1, :])
        lt = half < u
        ybins = (jnp.where(go, float(NBINS // 2), 0.0)
                 + jnp.sum(lt.astype(jnp.float32), axis=0, keepdims=True))
        offset = jnp.maximum(
            jnp.max(jnp.where(lt, half, 0.0), axis=0, keepdims=True),
            jnp.where(go, C32, 0.0))
        nxt = jnp.minimum(
            jnp.min(jnp.where(lt, BIG, half), axis=0, keepdims=True),
            jnp.where(go, Qnorm, C32))
        slope64 = (nxt - offset) * float(NBINS)
        out_ref[t:t + 1, :] = (u - offset) / slope64 + ybins * (1.0 / NBINS)
        inv_prod = inv_prod * (Qnorm / slope64)

    out_ref[TRANS:TRANS + 1, :] = inv_prod


def kernel(y, W1, b1, W2, b2, W3, b3):
    B = y.shape[0]
    grid = (B // BLOCK,)
    tri = (jnp.arange(NBINS)[:, None] >= jnp.arange(NBINS)[None, :]
           ).astype(jnp.float32)
    tric = (tri[None, :, :] * jnp.exp(b3).reshape(TRANS, 1, NBINS)
            ).reshape(TRANS * NBINS, NBINS)
    o9 = pl.pallas_call(
        _coupling_kernel,
        grid=grid,
        in_specs=[
            pl.BlockSpec((FLOW + 1, BLOCK), lambda i: (0, i)),
            pl.BlockSpec((HID, PASS), lambda i: (0, 0)),
            pl.BlockSpec((HID, 1), lambda i: (0, 0)),
            pl.BlockSpec((HID, HID), lambda i: (0, 0)),
            pl.BlockSpec((HID, 1), lambda i: (0, 0)),
            pl.BlockSpec((TRANS * NBINS, HID), lambda i: (0, 0)),
            pl.BlockSpec((TRANS * NBINS, NBINS), lambda i: (0, 0)),
        ],
        out_specs=pl.BlockSpec((TRANS + 1, BLOCK), lambda i: (0, i)),
        out_shape=jax.ShapeDtypeStruct((TRANS + 1, B), jnp.float32),
    )(y.T, W1.T, b1[:, None], W2.T, b2[:, None], W3.T, tric)
    return jnp.concatenate([y[:, :PASS], o9.T], axis=1)


# speedup vs baseline: 1.5986x; 1.0145x over previous
import jax
import jax.numpy as jnp
from jax.experimental import pallas as pl

PASS = 8
FLOW = 16
TRANS = FLOW - PASS
NBINS = 64
HID = 64
BLOCK = 8192
BIG = 3.0e38


def _coupling_kernel(yT_ref, W1T_ref, b1_ref, W2T_ref, b2_ref, W3T_ref,
                     tric_ref, out_ref):
    yT = yT_ref[...]
    yAT = yT[:PASS, :]

    h = jnp.maximum(
        jnp.dot(W1T_ref[...], yAT, preferred_element_type=jnp.float32)
        + b1_ref[...], 0.0)
    h = jnp.maximum(
        jnp.dot(W2T_ref[...], h, preferred_element_type=jnp.float32)
        + b2_ref[...], 0.0)
    inv_prod = yT[FLOW:FLOW + 1, :]
    for t in range(TRANS):
        Et = jnp.exp(jnp.dot(W3T_ref[t * NBINS:(t + 1) * NBINS, :], h,
                             preferred_element_type=jnp.float32))
        Qs = jnp.dot(tric_ref[t * NBINS:(t + 1) * NBINS, :], Et,
                     preferred_element_type=jnp.float32)
        Qnorm = Qs[NBINS - 1:NBINS, :]
        u = yT[PASS + t:PASS + t + 1, :] * Qnorm
        C32 = Qs[NBINS // 2 - 1:NBINS // 2, :]
        go = C32 < u
        half = jnp.where(go, Qs[NBINS // 2:NBINS - 1, :],
                         Qs[:NBINS // 2 - 1, :])
        lt = half < u
        ybins = (jnp.where(go, float(NBINS // 2), 0.0)
                 + jnp.sum(lt.astype(jnp.float32), axis=0, keepdims=True))
        offset = jnp.maximum(
            jnp.max(jnp.where(lt, half, 0.0), axis=0, keepdims=True),
            jnp.where(go, C32, 0.0))
        nxt = jnp.minimum(
            jnp.min(jnp.where(lt, BIG, half), axis=0, keepdims=True),
            jnp.where(go, Qnorm, C32))
        slope64 = (nxt - offset) * float(NBINS)
        out_ref[t:t + 1, :] = (u - offset) / slope64 + ybins * (1.0 / NBINS)
        inv_prod = inv_prod * (Qnorm / slope64)

    out_ref[TRANS:TRANS + 1, :] = inv_prod


def kernel(y, W1, b1, W2, b2, W3, b3):
    B = y.shape[0]
    grid = (B // BLOCK,)
    tri = (jnp.arange(NBINS)[:, None] >= jnp.arange(NBINS)[None, :]
           ).astype(jnp.float32)
    tric = (tri[None, :, :] * jnp.exp(b3).reshape(TRANS, 1, NBINS)
            ).reshape(TRANS * NBINS, NBINS)
    o9 = pl.pallas_call(
        _coupling_kernel,
        grid=grid,
        in_specs=[
            pl.BlockSpec((FLOW + 1, BLOCK), lambda i: (0, i)),
            pl.BlockSpec((HID, PASS), lambda i: (0, 0)),
            pl.BlockSpec((HID, 1), lambda i: (0, 0)),
            pl.BlockSpec((HID, HID), lambda i: (0, 0)),
            pl.BlockSpec((HID, 1), lambda i: (0, 0)),
            pl.BlockSpec((TRANS * NBINS, HID), lambda i: (0, 0)),
            pl.BlockSpec((TRANS * NBINS, NBINS), lambda i: (0, 0)),
        ],
        out_specs=pl.BlockSpec((TRANS + 1, BLOCK), lambda i: (0, i)),
        out_shape=jax.ShapeDtypeStruct((TRANS + 1, B), jnp.float32),
    )(y.T, W1.T, b1[:, None], W2.T, b2[:, None], W3.T, tric)
    return jnp.concatenate([y[:, :PASS], o9.T], axis=1)
